# baseline (speedup 1.0000x reference)
import functools
import jax
import jax.numpy as jnp
from jax import lax
from jax.experimental import pallas as pl
from jax.experimental.pallas import tpu as pltpu
from jax.experimental.pallas import tpu_sc as plsc

NUM_CLASSES = 60
NUM_INSTANCE = 1024
MEM_DIM = 256
FC_DIM = 1024
N = 16384
NW = 32


def _wid():
    return lax.axis_index("s") * 2 + lax.axis_index("c")


def _iota16():
    return lax.iota(jnp.int32, 16)


def _extract(vec, lane):
    return jnp.max(jnp.where(_iota16() == lane, vec, 0))


def _lower_bound(cls_v, q):
    lo = jnp.zeros((16,), jnp.int32)
    hi = jnp.full((16,), N, jnp.int32)

    def step(_, carry):
        lo, hi = carry
        mid = lax.div(lo + hi, 2)
        v = plsc.load_gather(cls_v, [jnp.minimum(mid, N - 1)])
        upd = lo < hi
        lt = v < q
        lo = jnp.where(upd & lt, mid + 1, lo)
        hi = jnp.where(upd & (~lt), mid, hi)
        return lo, hi

    lo, _ = lax.fori_loop(0, 15, step, (lo, hi))
    return lo


def _class_bounds(cls_v, c):
    lo = _lower_bound(cls_v, c + jnp.minimum(_iota16(), 1))
    first = _extract(lo, 0)
    nxt = _extract(lo, 1)
    return first, nxt - first


def _copy_rows(length, src_ref, dst_ref, buf2, idx2, gsem, ssem,
               idx_of, ch):
    n = pl.cdiv(length, ch)

    def gather_of(b):
        return pltpu.make_async_copy(src_ref.at[idx2.at[b, 0]], buf2.at[b],
                                     gsem.at[b])

    def scatter_of(b):
        return pltpu.make_async_copy(buf2.at[b], dst_ref.at[idx2.at[b, 1]],
                                     ssem.at[b])

    def issue_gather(k, b):
        for j in range(ch // 16):
            s = jnp.minimum(k * ch + j * 16 + _iota16(), length - 1)
            src, dst = idx_of(s)
            idx2[b, 0, pl.ds(j * 16, 16)] = src
            idx2[b, 1, pl.ds(j * 16, 16)] = dst
        gather_of(b).start()

    def body(k, _):
        b = lax.rem(k, 2)
        nb = 1 - b

        @pl.when(k + 1 < n)
        def _():
            @pl.when(k >= 1)
            def _():
                scatter_of(nb).wait()

            issue_gather(k + 1, nb)

        gather_of(b).wait()
        scatter_of(b).start()
        return 0

    @pl.when(length > 0)
    def _():
        issue_gather(0, 0)
        lax.fori_loop(0, n, body, 0)

        @pl.when(n >= 2)
        def _():
            scatter_of(lax.rem(n, 2)).wait()

        scatter_of(lax.rem(n - 1, 2)).wait()


def _build_tbl(cls_v, tbl_v):
    for q in range(4):
        tbl_v[pl.ds(q * 16, 16)] = _lower_bound(cls_v, q * 16 + _iota16())


_EPW = N // NW


def _fresh_elems(w, cls_v, tbl_v, feat, out, buf2, idx2, gsem, ssem, ch):
    e0 = w * _EPW

    def idx_of(s):
        i = e0 + s
        c = plsc.load_gather(cls_v, [i])
        f = plsc.load_gather(tbl_v, [c])
        nx = plsc.load_gather(tbl_v, [c + 1])
        slot = lax.rem(i - f, NUM_INSTANCE)
        src = f + slot + NUM_INSTANCE * lax.div(nx - f - 1 - slot,
                                                NUM_INSTANCE)
        return src, c * NUM_INSTANCE + slot

    _copy_rows(_EPW, feat, out, buf2, idx2, gsem, ssem, idx_of, ch)


def _roi_body(rf, rc, out, cls_v, tbl_v, buf2, idx2, gsem, ssem):
    w = _wid()
    pltpu.sync_copy(rc, cls_v)
    _build_tbl(cls_v, tbl_v)
    _fresh_elems(w, cls_v, tbl_v, rf, out, buf2, idx2, gsem, ssem, 48)


_PORT = 8 * NUM_INSTANCE
_ECH = 512


def _small_common(cls_v, c0, e0, e1, tbl_v, src_stage, scatter_cols):
    nch = pl.cdiv(jnp.maximum(e1 - e0, 0), _ECH)

    def chunk(k, _):
        start = e0 + k * _ECH
        astart = jnp.minimum(lax.div(start, 8) * 8, N - (_ECH + 8))
        astart = pl.multiple_of(astart, 8)
        src_stage(astart)
        rem = jnp.minimum(e1 - start, _ECH)

        def grp(g, _):
            ii = start + g * 16 + _iota16()
            valid = (g * 16 + _iota16()) < rem
            i_c = jnp.minimum(ii, e1 - 1)
            cv = plsc.load_gather(cls_v, [i_c])
            f = plsc.load_gather(tbl_v, [cv - c0])
            nx = plsc.load_gather(tbl_v, [cv - c0 + 1])
            slot = lax.rem(i_c - f, NUM_INSTANCE)
            lastw = (i_c + NUM_INSTANCE) >= nx
            dstrow = (cv - c0) * NUM_INSTANCE + slot
            scatter_cols(i_c - astart, dstrow, valid & lastw)
            return 0

        lax.fori_loop(0, pl.cdiv(rem, 16), grp, 0)
        return 0

    lax.fori_loop(0, nch, chunk, 0)


def _small_work(w, clsp_v, clsr_v, tbl_v, dout, dchunk, sout, schunk,
                pd, ps, rd, rs, pdm, psm, rdm, rsm, odm, osm, ordm, orsm):
    role = lax.div(w, 8)
    sub = lax.rem(w, 8)
    c0 = jnp.minimum(sub * 8, NUM_CLASSES - 8)
    rb = c0 * NUM_INSTANCE

    def setup(cls_v):
        lo = _lower_bound(cls_v, c0 + _iota16())
        tbl_v[pl.ds(0, 16)] = lo
        return _extract(lo, 0), _extract(lo, 8)

    def delta_role(cls_v, feat, mem, out):
        e0, e1 = setup(cls_v)
        pltpu.sync_copy(mem.at[pl.ds(pl.multiple_of(rb * 4, 8), _PORT * 4)],
                        dout)

        def stage(astart):
            pltpu.sync_copy(
                feat.at[pl.ds(pl.multiple_of(astart * 4, 8), (_ECH + 8) * 4)],
                dchunk)

        def scat(srow, drow, m):
            for j in range(4):
                val = plsc.load_gather(dchunk, [srow * 4 + j])
                plsc.store_scatter(dout, [drow * 4 + j], val, mask=m)

        _small_common(cls_v, c0, e0, e1, tbl_v, stage, scat)
        pltpu.sync_copy(dout,
                        out.at[pl.ds(pl.multiple_of(rb * 4, 8), _PORT * 4)])

    def scale_role(cls_v, feat, mem, out):
        e0, e1 = setup(cls_v)
        pltpu.sync_copy(mem.at[pl.ds(pl.multiple_of(rb, 8), _PORT)], sout)

        def stage(astart):
            pltpu.sync_copy(feat.at[pl.ds(astart, _ECH + 8)], schunk)

        def scat(srow, drow, m):
            val = plsc.load_gather(schunk, [srow])
            plsc.store_scatter(sout, [drow], val, mask=m)

        _small_common(cls_v, c0, e0, e1, tbl_v, stage, scat)
        pltpu.sync_copy(sout, out.at[pl.ds(pl.multiple_of(rb, 8), _PORT)])

    @pl.when(role == 0)
    def _():
        delta_role(clsp_v, pd, pdm, odm)

    @pl.when(role == 1)
    def _():
        delta_role(clsr_v, rd, rdm, ordm)

    @pl.when(role == 2)
    def _():
        scale_role(clsp_v, ps, psm, osm)

    @pl.when(role == 3)
    def _():
        scale_role(clsr_v, rs, rsm, orsm)


def _rest_body(pc, rc, pf, pd, ps, rd, rs, pdm, psm, rdm, rsm, om,
               odm, osm, ordm, orsm, otbl,
               clsp_v, clsr_v, buf2, idx2, tblp_v, tbl_v, dout, dchunk,
               sout, schunk, gsem, ssem):
    w = _wid()
    pltpu.sync_copy(pc, clsp_v)
    pltpu.sync_copy(rc, clsr_v)
    _build_tbl(clsp_v, tblp_v)

    @pl.when(w == 0)
    def _():
        _build_tbl(clsr_v, tblp_v)
        pltpu.sync_copy(tblp_v, otbl)
        _build_tbl(clsp_v, tblp_v)

    _fresh_elems(w, clsp_v, tblp_v, pf, om, buf2, idx2, gsem, ssem, 64)
    _small_work(w, clsp_v, clsr_v, tbl_v, dout, dchunk, sout, schunk,
                pd, ps, rd, rs, pdm, psm, rdm, rsm, odm, osm, ordm, orsm)


def _tc_copy(x, br):
    def body(s_ref, d_ref):
        d_ref[...] = s_ref[...]

    rows, cols = x.shape
    return pl.pallas_call(
        body,
        grid=(rows // br,),
        in_specs=[pl.BlockSpec((br, cols), lambda i: (i, 0))],
        out_specs=pl.BlockSpec((br, cols), lambda i: (i, 0)),
        out_shape=jax.ShapeDtypeStruct(x.shape, x.dtype),
        cost_estimate=pl.CostEstimate(
            flops=0, transcendentals=0, bytes_accessed=8 * rows * cols),
    )(x)


def _tc_copy_skip(x, tbl, br):
    rows, cols = x.shape
    nsk = NUM_INSTANCE // br
    grid = rows // br
    NB, LA = 4, 3

    def body(tbl_ref, src, dst, buf, issued, gsem, ssem):
        i = pl.program_id(0)

        def skip(ii):
            c = lax.div(ii, nsk)
            j = lax.rem(ii, nsk)
            cnt1 = jnp.minimum(tbl_ref[c + 1] - tbl_ref[c], NUM_INSTANCE)
            return (j + 1) * br <= cnt1

        def g_copy(ii, sl):
            return pltpu.make_async_copy(src.at[pl.ds(ii * br, br)],
                                         buf.at[sl], gsem.at[sl])

        def s_copy(ii, sl):
            return pltpu.make_async_copy(buf.at[sl],
                                         dst.at[pl.ds(ii * br, br)],
                                         ssem.at[sl])

        def maybe_gather(ii):
            @pl.when(ii < grid)
            def _():
                sl = lax.rem(ii, NB)

                @pl.when(issued[sl] == 1)
                def _():
                    s_copy(ii, sl).wait()
                    issued[sl] = 0

                @pl.when(~skip(ii))
                def _():
                    g_copy(ii, sl).start()

        @pl.when(i == 0)
        def _():
            for sl in range(NB):
                issued[sl] = 0
            for k in range(LA):
                maybe_gather(k)

        maybe_gather(i + LA)

        @pl.when(~skip(i))
        def _():
            sl = lax.rem(i, NB)
            g_copy(i, sl).wait()
            s_copy(i, sl).start()
            issued[sl] = 1

        @pl.when(i == grid - 1)
        def _():
            for sl in range(NB):
                @pl.when(issued[sl] == 1)
                def _():
                    s_copy(i, sl).wait()

    return pl.pallas_call(
        body,
        grid_spec=pltpu.PrefetchScalarGridSpec(
            num_scalar_prefetch=1,
            grid=(grid,),
            in_specs=[pl.BlockSpec(memory_space=pl.ANY)],
            out_specs=pl.BlockSpec(memory_space=pl.ANY),
            scratch_shapes=[
                pltpu.VMEM((NB, br, cols), jnp.float32),
                pltpu.SMEM((NB,), jnp.int32),
                pltpu.SemaphoreType.DMA((NB,)),
                pltpu.SemaphoreType.DMA((NB,)),
            ],
        ),
        out_shape=jax.ShapeDtypeStruct(x.shape, x.dtype),
        compiler_params=pltpu.CompilerParams(
            dimension_semantics=("arbitrary",)),
        cost_estimate=pl.CostEstimate(
            flops=0, transcendentals=0, bytes_accessed=6 * rows * cols),
    )(tbl, x)


def kernel(prop_feature, prop_deltas, prop_scale, prop_class,
           roi_feature, roi_deltas, roi_scale, roi_class,
           prop_mem, prop_delta_mem, prop_scale_mem,
           roi_mem, roi_delta_mem, roi_scale_mem):
    R = NUM_CLASSES * NUM_INSTANCE
    mesh = plsc.VectorSubcoreMesh(core_axis_name="c", subcore_axis_name="s")
    params = pltpu.CompilerParams(needs_layout_passes=False)

    po = jax.new_ref(_tc_copy(prop_mem.reshape(R, MEM_DIM), 2048))

    roi_k = functools.partial(
        pl.kernel, mesh=mesh,
        out_type=(),
        cost_estimate=pl.CostEstimate(
            flops=0, transcendentals=0, bytes_accessed=8 * R * FC_DIM // 4),
        scratch_types=[
            pltpu.VMEM((N,), jnp.int32),
            pltpu.VMEM((64,), jnp.int32),
            pltpu.VMEM((2, 48, FC_DIM), jnp.float32),
            pltpu.VMEM((2, 2, 48), jnp.int32),
            pltpu.SemaphoreType.DMA((2,)),
            pltpu.SemaphoreType.DMA((2,)),
        ],
        compiler_params=params,
    )(_roi_body)

    rest_k = functools.partial(
        pl.kernel, mesh=mesh,
        cost_estimate=pl.CostEstimate(
            flops=0, transcendentals=0, bytes_accessed=64 * 1024 * 1024),
        out_type=(
            jax.ShapeDtypeStruct((R * 4,), jnp.float32),
            jax.ShapeDtypeStruct((R,), jnp.float32),
            jax.ShapeDtypeStruct((R * 4,), jnp.float32),
            jax.ShapeDtypeStruct((R,), jnp.float32),
            jax.ShapeDtypeStruct((64,), jnp.int32),
        ),
        scratch_types=[
            pltpu.VMEM((N,), jnp.int32),
            pltpu.VMEM((N,), jnp.int32),
            pltpu.VMEM((2, 64, MEM_DIM), jnp.float32),
            pltpu.VMEM((2, 2, 64), jnp.int32),
            pltpu.VMEM((64,), jnp.int32),
            pltpu.VMEM((16,), jnp.int32),
            pltpu.VMEM((_PORT * 4,), jnp.float32),
            pltpu.VMEM(((_ECH + 8) * 4,), jnp.float32),
            pltpu.VMEM((_PORT,), jnp.float32),
            pltpu.VMEM((_ECH + 8,), jnp.float32),
            pltpu.SemaphoreType.DMA((2,)),
            pltpu.SemaphoreType.DMA((2,)),
        ],
        compiler_params=params,
    )(_rest_body)
    odm, osm, ordm, orsm, rtbl = rest_k(
        prop_class, roi_class, prop_feature,
        prop_deltas.reshape(N * 4), prop_scale,
        roi_deltas.reshape(N * 4), roi_scale,
        prop_delta_mem.reshape(R * 4), prop_scale_mem.reshape(R),
        roi_delta_mem.reshape(R * 4), roi_scale_mem.reshape(R),
        po)
    ro = jax.new_ref(_tc_copy_skip(roi_mem.reshape(R, FC_DIM), rtbl, 128))
    roi_k(roi_feature, roi_class, ro)
    om = po[...]
    roi_out = ro[...]

    shape3 = (NUM_CLASSES, NUM_INSTANCE)
    return (om.reshape(*shape3, MEM_DIM), odm.reshape(*shape3, 4),
            osm.reshape(*shape3),
            roi_out.reshape(*shape3, FC_DIM), ordm.reshape(*shape3, 4),
            orsm.reshape(*shape3))


# speedup vs baseline: 1.3800x; 1.3800x over previous
import functools
import jax
import jax.numpy as jnp
from jax import lax
from jax.experimental import pallas as pl
from jax.experimental.pallas import tpu as pltpu
from jax.experimental.pallas import tpu_sc as plsc

NUM_CLASSES = 60
NUM_INSTANCE = 1024
MEM_DIM = 256
FC_DIM = 1024
N = 16384
NW = 32


def _wid():
    return lax.axis_index("s") * 2 + lax.axis_index("c")


def _iota16():
    return lax.iota(jnp.int32, 16)


def _extract(vec, lane):
    return jnp.max(jnp.where(_iota16() == lane, vec, 0))


def _lower_bound(cls_v, q):
    lo = jnp.zeros((16,), jnp.int32)
    hi = jnp.full((16,), N, jnp.int32)

    def step(_, carry):
        lo, hi = carry
        mid = lax.div(lo + hi, 2)
        v = plsc.load_gather(cls_v, [jnp.minimum(mid, N - 1)])
        upd = lo < hi
        lt = v < q
        lo = jnp.where(upd & lt, mid + 1, lo)
        hi = jnp.where(upd & (~lt), mid, hi)
        return lo, hi

    lo, _ = lax.fori_loop(0, 15, step, (lo, hi))
    return lo


def _copy_rows(length, src_ref, dst_ref, buf2, idx2, gsem, ssem,
               idx_of, ch):
    n = pl.cdiv(length, ch)

    def gather_of(b):
        return pltpu.make_async_copy(src_ref.at[idx2.at[b, 0]], buf2.at[b],
                                     gsem.at[b])

    def scatter_of(b):
        return pltpu.make_async_copy(buf2.at[b], dst_ref.at[idx2.at[b, 1]],
                                     ssem.at[b])

    def issue_gather(k, b):
        for j in range(ch // 16):
            s = jnp.minimum(k * ch + j * 16 + _iota16(), length - 1)
            src, dst = idx_of(s)
            idx2[b, 0, pl.ds(j * 16, 16)] = src
            idx2[b, 1, pl.ds(j * 16, 16)] = dst
        gather_of(b).start()

    def body(k, _):
        b = lax.rem(k, 2)
        nb = 1 - b

        @pl.when(k + 1 < n)
        def _():
            @pl.when(k >= 1)
            def _():
                scatter_of(nb).wait()

            issue_gather(k + 1, nb)

        gather_of(b).wait()
        scatter_of(b).start()
        return 0

    @pl.when(length > 0)
    def _():
        issue_gather(0, 0)
        lax.fori_loop(0, n, body, 0)

        @pl.when(n >= 2)
        def _():
            scatter_of(lax.rem(n, 2)).wait()

        scatter_of(lax.rem(n - 1, 2)).wait()


def _build_tbl(cls_v, tbl_v):
    for q in range(4):
        tbl_v[pl.ds(q * 16, 16)] = _lower_bound(cls_v, q * 16 + _iota16())


_EPW = N // NW


def _fresh_elems(w, cls_v, tbl_v, feat, out, buf2, idx2, gsem, ssem, ch):
    e0 = w * _EPW

    def idx_of(s):
        i = e0 + s
        c = plsc.load_gather(cls_v, [i])
        f = plsc.load_gather(tbl_v, [c])
        nx = plsc.load_gather(tbl_v, [c + 1])
        slot = lax.rem(i - f, NUM_INSTANCE)
        src = f + slot + NUM_INSTANCE * lax.div(nx - f - 1 - slot,
                                                NUM_INSTANCE)
        return src, c * NUM_INSTANCE + slot

    _copy_rows(_EPW, feat, out, buf2, idx2, gsem, ssem, idx_of, ch)


def _roi_body(rf, rc, out, cls_v, tbl_v, buf2, idx2, gsem, ssem):
    w = _wid()
    pltpu.sync_copy(rc, cls_v)
    _build_tbl(cls_v, tbl_v)
    _fresh_elems(w, cls_v, tbl_v, rf, out, buf2, idx2, gsem, ssem, 48)


_PORT = 8 * NUM_INSTANCE
_ECH = 512


def _small_common(cls_v, c0, e0, e1, tbl_v, src_stage, scatter_cols):
    nch = pl.cdiv(jnp.maximum(e1 - e0, 0), _ECH)

    def chunk(k, _):
        start = e0 + k * _ECH
        astart = jnp.minimum(lax.div(start, 8) * 8, N - (_ECH + 8))
        astart = pl.multiple_of(astart, 8)
        src_stage(astart)
        rem = jnp.minimum(e1 - start, _ECH)

        def grp(g, _):
            ii = start + g * 16 + _iota16()
            valid = (g * 16 + _iota16()) < rem
            i_c = jnp.minimum(ii, e1 - 1)
            cv = plsc.load_gather(cls_v, [i_c])
            f = plsc.load_gather(tbl_v, [cv - c0])
            nx = plsc.load_gather(tbl_v, [cv - c0 + 1])
            slot = lax.rem(i_c - f, NUM_INSTANCE)
            lastw = (i_c + NUM_INSTANCE) >= nx
            dstrow = (cv - c0) * NUM_INSTANCE + slot
            scatter_cols(i_c - astart, dstrow, valid & lastw)
            return 0

        lax.fori_loop(0, pl.cdiv(rem, 16), grp, 0)
        return 0

    lax.fori_loop(0, nch, chunk, 0)


def _small_work(w, clsp_v, clsr_v, tbl_v, dout, dchunk, sout, schunk,
                pd, ps, rd, rs, pdm, psm, rdm, rsm, odm, osm, ordm, orsm):
    role = lax.div(w, 8)
    sub = lax.rem(w, 8)
    c0 = jnp.minimum(sub * 8, NUM_CLASSES - 8)
    rb = c0 * NUM_INSTANCE

    def setup(cls_v):
        lo = _lower_bound(cls_v, c0 + _iota16())
        tbl_v[pl.ds(0, 16)] = lo
        return _extract(lo, 0), _extract(lo, 8)

    def delta_role(cls_v, feat, mem, out):
        e0, e1 = setup(cls_v)
        pltpu.sync_copy(mem.at[pl.ds(pl.multiple_of(rb * 4, 8), _PORT * 4)],
                        dout)

        def stage(astart):
            pltpu.sync_copy(
                feat.at[pl.ds(pl.multiple_of(astart * 4, 8), (_ECH + 8) * 4)],
                dchunk)

        def scat(srow, drow, m):
            for j in range(4):
                val = plsc.load_gather(dchunk, [srow * 4 + j])
                plsc.store_scatter(dout, [drow * 4 + j], val, mask=m)

        _small_common(cls_v, c0, e0, e1, tbl_v, stage, scat)
        pltpu.sync_copy(dout,
                        out.at[pl.ds(pl.multiple_of(rb * 4, 8), _PORT * 4)])

    def scale_role(cls_v, feat, mem, out):
        e0, e1 = setup(cls_v)
        pltpu.sync_copy(mem.at[pl.ds(pl.multiple_of(rb, 8), _PORT)], sout)

        def stage(astart):
            pltpu.sync_copy(feat.at[pl.ds(astart, _ECH + 8)], schunk)

        def scat(srow, drow, m):
            val = plsc.load_gather(schunk, [srow])
            plsc.store_scatter(sout, [drow], val, mask=m)

        _small_common(cls_v, c0, e0, e1, tbl_v, stage, scat)
        pltpu.sync_copy(sout, out.at[pl.ds(pl.multiple_of(rb, 8), _PORT)])

    @pl.when(role == 0)
    def _():
        delta_role(clsp_v, pd, pdm, odm)

    @pl.when(role == 1)
    def _():
        delta_role(clsr_v, rd, rdm, ordm)

    @pl.when(role == 2)
    def _():
        scale_role(clsp_v, ps, psm, osm)

    @pl.when(role == 3)
    def _():
        scale_role(clsr_v, rs, rsm, orsm)


def _rest_body(pc, rc, pf, pd, ps, rd, rs, pdm, psm, rdm, rsm, om,
               odm, osm, ordm, orsm,
               clsp_v, clsr_v, buf2, idx2, tblp_v, tbl_v, dout, dchunk,
               sout, schunk, gsem, ssem):
    w = _wid()
    pltpu.sync_copy(pc, clsp_v)
    pltpu.sync_copy(rc, clsr_v)
    _build_tbl(clsp_v, tblp_v)
    _fresh_elems(w, clsp_v, tblp_v, pf, om, buf2, idx2, gsem, ssem, 64)
    _small_work(w, clsp_v, clsr_v, tbl_v, dout, dchunk, sout, schunk,
                pd, ps, rd, rs, pdm, psm, rdm, rsm, odm, osm, ordm, orsm)


def _tc_copy(x, br):
    def body(s_ref, d_ref):
        d_ref[...] = s_ref[...]

    rows, cols = x.shape
    return pl.pallas_call(
        body,
        grid=(rows // br,),
        in_specs=[pl.BlockSpec((br, cols), lambda i: (i, 0))],
        out_specs=pl.BlockSpec((br, cols), lambda i: (i, 0)),
        out_shape=jax.ShapeDtypeStruct(x.shape, x.dtype),
        cost_estimate=pl.CostEstimate(
            flops=0, transcendentals=0, bytes_accessed=8 * rows * cols),
    )(x)


def kernel(prop_feature, prop_deltas, prop_scale, prop_class,
           roi_feature, roi_deltas, roi_scale, roi_class,
           prop_mem, prop_delta_mem, prop_scale_mem,
           roi_mem, roi_delta_mem, roi_scale_mem):
    R = NUM_CLASSES * NUM_INSTANCE
    mesh = plsc.VectorSubcoreMesh(core_axis_name="c", subcore_axis_name="s")
    params = pltpu.CompilerParams(needs_layout_passes=False)

    po = jax.new_ref(_tc_copy(prop_mem.reshape(R, MEM_DIM), 2048))

    roi_k = functools.partial(
        pl.kernel, mesh=mesh,
        out_type=(),
        cost_estimate=pl.CostEstimate(
            flops=0, transcendentals=0, bytes_accessed=8 * R * FC_DIM // 4),
        scratch_types=[
            pltpu.VMEM((N,), jnp.int32),
            pltpu.VMEM((64,), jnp.int32),
            pltpu.VMEM((2, 48, FC_DIM), jnp.float32),
            pltpu.VMEM((2, 2, 48), jnp.int32),
            pltpu.SemaphoreType.DMA((2,)),
            pltpu.SemaphoreType.DMA((2,)),
        ],
        compiler_params=params,
    )(_roi_body)

    rest_k = functools.partial(
        pl.kernel, mesh=mesh,
        cost_estimate=pl.CostEstimate(
            flops=0, transcendentals=0, bytes_accessed=64 * 1024 * 1024),
        out_type=(
            jax.ShapeDtypeStruct((R * 4,), jnp.float32),
            jax.ShapeDtypeStruct((R,), jnp.float32),
            jax.ShapeDtypeStruct((R * 4,), jnp.float32),
            jax.ShapeDtypeStruct((R,), jnp.float32),
        ),
        scratch_types=[
            pltpu.VMEM((N,), jnp.int32),
            pltpu.VMEM((N,), jnp.int32),
            pltpu.VMEM((2, 64, MEM_DIM), jnp.float32),
            pltpu.VMEM((2, 2, 64), jnp.int32),
            pltpu.VMEM((64,), jnp.int32),
            pltpu.VMEM((16,), jnp.int32),
            pltpu.VMEM((_PORT * 4,), jnp.float32),
            pltpu.VMEM(((_ECH + 8) * 4,), jnp.float32),
            pltpu.VMEM((_PORT,), jnp.float32),
            pltpu.VMEM((_ECH + 8,), jnp.float32),
            pltpu.SemaphoreType.DMA((2,)),
            pltpu.SemaphoreType.DMA((2,)),
        ],
        compiler_params=params,
    )(_rest_body)
    odm, osm, ordm, orsm = rest_k(
        prop_class, roi_class, prop_feature,
        prop_deltas.reshape(N * 4), prop_scale,
        roi_deltas.reshape(N * 4), roi_scale,
        prop_delta_mem.reshape(R * 4), prop_scale_mem.reshape(R),
        roi_delta_mem.reshape(R * 4), roi_scale_mem.reshape(R),
        po)
    ro = jax.new_ref(_tc_copy(roi_mem.reshape(R, FC_DIM), 1024))
    roi_k(roi_feature, roi_class, ro)
    om = po[...]
    roi_out = ro[...]

    shape3 = (NUM_CLASSES, NUM_INSTANCE)
    return (om.reshape(*shape3, MEM_DIM), odm.reshape(*shape3, 4),
            osm.reshape(*shape3),
            roi_out.reshape(*shape3, FC_DIM), ordm.reshape(*shape3, 4),
            orsm.reshape(*shape3))
